# baseline (speedup 1.0000x reference)
import functools

import jax
import jax.numpy as jnp
from jax import lax
from jax.experimental import pallas as pl
from jax.experimental.pallas import tpu as pltpu
from jax.experimental.pallas import tpu_sc as plsc

FEATURE_DIM = 128
BOARD_SIZE = 15
PCODE_DIM = 2380
EMBED_DIM = 2 * (PCODE_DIM + 1)
CELL_DIM = BOARD_SIZE * BOARD_SIZE

NUM_CORES = 2
NUM_SUBCORES = 16
NUM_WORKERS = NUM_CORES * NUM_SUBCORES
LANES = 16

W = 96
CHUNKS = W // LANES


def _sc_gather_sum(idx_in, w_pcode, w_board, n_rows):
    per_w = n_rows // NUM_WORKERS
    wins = per_w // W
    mesh = plsc.VectorSubcoreMesh(core_axis_name="c", subcore_axis_name="s")

    @functools.partial(
        pl.kernel,
        mesh=mesh,
        out_type=jax.ShapeDtypeStruct((n_rows, FEATURE_DIM), jnp.float32),
        scratch_types=[
            pltpu.VMEM((4, W), jnp.int32),
            pltpu.VMEM((4, W), jnp.int32),
            pltpu.VMEM((W, FEATURE_DIM), jnp.float32),
            pltpu.VMEM((W, FEATURE_DIM), jnp.float32),
            pltpu.VMEM((W, FEATURE_DIM), jnp.float32),
            pltpu.VMEM((W, FEATURE_DIM), jnp.float32),
            pltpu.SemaphoreType.DMA,
        ],
    )
    def k(idx_hbm, wp_hbm, wb_hbm, out_hbm, raw_v, idx_v, g0, g1, g2, g3, sem):
        wid = lax.axis_index("s") * NUM_CORES + lax.axis_index("c")
        base0 = wid * per_w

        @pl.loop(0, wins)
        def _(w):
            base = base0 + w * W
            for j in range(4):
                pltpu.sync_copy(idx_hbm.at[pl.ds(j * n_rows + base, W)], raw_v.at[j])

            @pl.loop(0, CHUNKS)
            def _(j):
                sl = pl.ds(j * LANES, LANES)
                occupied = (raw_v[2, sl] + raw_v[3, sl]) > 0
                p0 = jnp.where(occupied, PCODE_DIM, raw_v[0, sl])
                p1 = jnp.where(
                    occupied, 2 * PCODE_DIM + 1, raw_v[1, sl] + (PCODE_DIM + 1)
                )
                pos = base + j * LANES + lax.iota(jnp.int32, LANES)
                coff = (pos % CELL_DIM) * EMBED_DIM
                idx_v[0, sl] = p0
                idx_v[1, sl] = p1
                idx_v[2, sl] = coff + p0
                idx_v[3, sl] = coff + p1

            c0 = pltpu.async_copy(wp_hbm.at[idx_v.at[0]], g0, sem)
            c1 = pltpu.async_copy(wp_hbm.at[idx_v.at[1]], g1, sem)
            c2 = pltpu.async_copy(wb_hbm.at[idx_v.at[2]], g2, sem)
            c3 = pltpu.async_copy(wb_hbm.at[idx_v.at[3]], g3, sem)
            c0.wait()
            c1.wait()
            c2.wait()
            c3.wait()

            @pl.loop(0, W)
            def _(r):
                for c in range(FEATURE_DIM // LANES):
                    sl = pl.ds(c * LANES, LANES)
                    g0[r, sl] = g0[r, sl] + (g1[r, sl] + (g2[r, sl] + g3[r, sl]))

            pltpu.sync_copy(g0, out_hbm.at[pl.ds(base, W)])

    return k(idx_in, w_pcode, w_board)


def _tc_transpose(s, batch):
    grp = 8

    def body(s_ref, o_ref):
        o_ref[...] = jnp.swapaxes(s_ref[...], 1, 2)

    return pl.pallas_call(
        body,
        grid=(batch // grp,),
        in_specs=[
            pl.BlockSpec((grp, CELL_DIM, FEATURE_DIM), lambda i: (i, 0, 0))
        ],
        out_specs=pl.BlockSpec((grp, FEATURE_DIM, CELL_DIM), lambda i: (i, 0, 0)),
        out_shape=jax.ShapeDtypeStruct((batch, FEATURE_DIM, CELL_DIM), jnp.float32),
    )(s)


def kernel(sparse_feature_input, sparse_feature_dim, board_input, W_pcode, W_board):
    del sparse_feature_dim
    batch = sparse_feature_input.shape[0]
    n_rows = batch * CELL_DIM
    pc = sparse_feature_input[:, 10:12].reshape(batch, 2, CELL_DIM)
    bd = board_input.reshape(batch, 2, CELL_DIM)
    idx_in = jnp.concatenate([pc, bd], axis=1)
    idx_in = jnp.transpose(idx_in, (1, 0, 2)).reshape(4 * n_rows)
    s = _sc_gather_sum(idx_in, W_pcode, W_board, n_rows)
    out = _tc_transpose(s.reshape(batch, CELL_DIM, FEATURE_DIM), batch)
    return out.reshape(batch, FEATURE_DIM, BOARD_SIZE, BOARD_SIZE)


# speedup vs baseline: 1.0053x; 1.0053x over previous
import functools

import jax
import jax.numpy as jnp
from jax import lax
from jax.experimental import pallas as pl
from jax.experimental.pallas import tpu as pltpu
from jax.experimental.pallas import tpu_sc as plsc

FEATURE_DIM = 128
BOARD_SIZE = 15
PCODE_DIM = 2380
EMBED_DIM = 2 * (PCODE_DIM + 1)
CELL_DIM = BOARD_SIZE * BOARD_SIZE

NUM_CORES = 2
NUM_SUBCORES = 16
NUM_WORKERS = NUM_CORES * NUM_SUBCORES
LANES = 16

W = 96
CHUNKS = W // LANES
RAW = 4 * W


def _sc_gather_sum(idx_in, w_pcode, w_board, n_rows):
    per_w = n_rows // NUM_WORKERS
    wins = per_w // W
    mesh = plsc.VectorSubcoreMesh(core_axis_name="c", subcore_axis_name="s")

    @functools.partial(
        pl.kernel,
        mesh=mesh,
        out_type=jax.ShapeDtypeStruct((n_rows, FEATURE_DIM), jnp.float32),
        scratch_types=[
            pltpu.VMEM((2, RAW), jnp.int32),
            pltpu.VMEM((2, 4, W), jnp.int32),
            pltpu.VMEM((2, 4, W, FEATURE_DIM), jnp.float32),
            pltpu.VMEM((2, W, FEATURE_DIM), jnp.float32),
            pltpu.SemaphoreType.DMA,
            pltpu.SemaphoreType.DMA,
            pltpu.SemaphoreType.DMA,
            pltpu.SemaphoreType.DMA,
        ],
    )
    def k(idx_hbm, wp_hbm, wb_hbm, out_hbm, raw_v, idx_v, g_v, o_v,
          gsA, gsB, osA, osB):
        wid = lax.axis_index("s") * NUM_CORES + lax.axis_index("c")
        base0 = wid * per_w

        def stage_fire(w, s, gsem):
            raw = raw_v.at[s]
            idx = idx_v.at[s]
            pltpu.sync_copy(idx_hbm.at[pl.ds((wid * wins + w) * RAW, RAW)], raw)
            base = base0 + w * W

            @pl.loop(0, CHUNKS)
            def _(j):
                sl = pl.ds(j * LANES, LANES)
                occupied = (raw[pl.ds(2 * W + j * LANES, LANES)]
                            + raw[pl.ds(3 * W + j * LANES, LANES)]) > 0
                p0 = jnp.where(occupied, PCODE_DIM, raw[pl.ds(j * LANES, LANES)])
                p1 = jnp.where(occupied, 2 * PCODE_DIM + 1,
                               raw[pl.ds(W + j * LANES, LANES)] + (PCODE_DIM + 1))
                pos = base + j * LANES + lax.iota(jnp.int32, LANES)
                coff = (pos % CELL_DIM) * EMBED_DIM
                idx[0, sl] = p0
                idx[1, sl] = p1
                idx[2, sl] = coff + p0
                idx[3, sl] = coff + p1
                pltpu.async_copy(wp_hbm.at[p0], g_v.at[s, 0, sl], gsem)
                pltpu.async_copy(wp_hbm.at[p1], g_v.at[s, 1, sl], gsem)
                pltpu.async_copy(wb_hbm.at[coff + p0], g_v.at[s, 2, sl], gsem)
                pltpu.async_copy(wb_hbm.at[coff + p1], g_v.at[s, 3, sl], gsem)

        def wait_gathers(s, gsem):
            for j, tab in enumerate((wp_hbm, wp_hbm, wb_hbm, wb_hbm)):
                for c in range(CHUNKS):
                    sl = pl.ds(c * LANES, LANES)
                    pltpu.make_async_copy(
                        tab.at[idx_v[s, j, sl]], g_v.at[s, j, sl], gsem
                    ).wait()

        def drain_out(s, osem):
            pltpu.make_async_copy(
                o_v.at[s], out_hbm.at[pl.ds(0, W)], osem
            ).wait()

        def accumulate(s):
            @pl.loop(0, W)
            def _(r):
                for c in range(FEATURE_DIM // LANES):
                    sl = pl.ds(c * LANES, LANES)
                    o_v[s, r, sl] = (g_v[s, 0, r, sl] + g_v[s, 1, r, sl]) + (
                        g_v[s, 2, r, sl] + g_v[s, 3, r, sl])

        def fire_out(w, s, osem):
            base = base0 + w * W
            pltpu.async_copy(o_v.at[s], out_hbm.at[pl.ds(base, W)], osem)

        def step(w, s, gsem, osem, drain_pred, last=False):
            wait_gathers(s, gsem)
            if drain_pred is True:
                drain_out(s, osem)
            elif drain_pred is not False:
                @pl.when(drain_pred)
                def _():
                    drain_out(s, osem)
            accumulate(s)
            fire_out(w, s, osem)
            if not last:
                @pl.when(w + 2 < wins)
                def _():
                    stage_fire(w + 2, s, gsem)

        stage_fire(0, 0, gsA)
        stage_fire(1, 1, gsB)

        @pl.loop(0, wins // 2)
        def _(t):
            step(2 * t, 0, gsA, osA, drain_pred=t > 0)
            step(2 * t + 1, 1, gsB, osB, drain_pred=t > 0)

        if wins % 2:
            step(wins - 1, 0, gsA, osA, drain_pred=True, last=True)
        drain_out(0, osA)
        drain_out(1, osB)

    return k(idx_in, w_pcode, w_board)


def _tc_transpose(s, batch):
    grp = 8

    def body(s_ref, o_ref):
        o_ref[...] = jnp.swapaxes(s_ref[...], 1, 2)

    return pl.pallas_call(
        body,
        grid=(batch // grp,),
        in_specs=[
            pl.BlockSpec((grp, CELL_DIM, FEATURE_DIM), lambda i: (i, 0, 0))
        ],
        out_specs=pl.BlockSpec((grp, FEATURE_DIM, CELL_DIM), lambda i: (i, 0, 0)),
        out_shape=jax.ShapeDtypeStruct((batch, FEATURE_DIM, CELL_DIM), jnp.float32),
    )(s)


def kernel(sparse_feature_input, sparse_feature_dim, board_input, W_pcode, W_board):
    del sparse_feature_dim
    batch = sparse_feature_input.shape[0]
    n_rows = batch * CELL_DIM
    pc = sparse_feature_input[:, 10:12].reshape(batch, 2, CELL_DIM)
    bd = board_input.reshape(batch, 2, CELL_DIM)
    idx_in = jnp.concatenate([pc, bd], axis=1)
    idx_flat = jnp.transpose(idx_in, (1, 0, 2)).reshape(4, n_rows)
    idx_win = jnp.transpose(
        idx_flat.reshape(4, n_rows // W, W), (1, 0, 2)
    ).reshape(-1)
    s = _sc_gather_sum(idx_win, W_pcode, W_board, n_rows)
    out = _tc_transpose(s.reshape(batch, CELL_DIM, FEATURE_DIM), batch)
    return out.reshape(batch, FEATURE_DIM, BOARD_SIZE, BOARD_SIZE)


# speedup vs baseline: 1.0066x; 1.0013x over previous
import functools

import jax
import jax.numpy as jnp
from jax import lax
from jax.experimental import pallas as pl
from jax.experimental.pallas import tpu as pltpu
from jax.experimental.pallas import tpu_sc as plsc

FEATURE_DIM = 128
BOARD_SIZE = 15
PCODE_DIM = 2380
EMBED_DIM = 2 * (PCODE_DIM + 1)
CELL_DIM = BOARD_SIZE * BOARD_SIZE

NUM_CORES = 2
NUM_SUBCORES = 16
NUM_WORKERS = NUM_CORES * NUM_SUBCORES
LANES = 16

W = 96
CHUNKS = W // LANES
RAW = 4 * W


def _sc_gather_sum(idx_in, w_pcode, w_board, n_rows):
    per_w = n_rows // NUM_WORKERS
    wins = per_w // W
    mesh = plsc.VectorSubcoreMesh(core_axis_name="c", subcore_axis_name="s")

    @functools.partial(
        pl.kernel,
        mesh=mesh,
        out_type=jax.ShapeDtypeStruct((n_rows, FEATURE_DIM), jnp.float32),
        scratch_types=[
            pltpu.SMEM((2, RAW), jnp.int32),
            pltpu.VMEM_SHARED((2, NUM_SUBCORES, RAW), jnp.int32),
            pltpu.VMEM((2, 4, W, FEATURE_DIM), jnp.float32),
            pltpu.VMEM((2, W, FEATURE_DIM), jnp.float32),
            pltpu.SemaphoreType.DMA,
            pltpu.SemaphoreType.DMA,
            pltpu.SemaphoreType.DMA,
            pltpu.SemaphoreType.DMA,
            pltpu.SemaphoreType.DMA,
            pltpu.SemaphoreType.DMA,
        ],
    )
    def k(idx_hbm, wp_hbm, wb_hbm, out_hbm, raw_s, raw_sp, g_v, o_v,
          gsA, gsB, osA, osB, rsA, rsB):
        wid = lax.axis_index("s") * NUM_CORES + lax.axis_index("c")
        sid = lax.axis_index("s")
        base0 = wid * per_w
        rsems = (rsA, rsB)

        def stage(w, s):
            pltpu.async_copy(
                idx_hbm.at[pl.ds((wid * wins + w) * RAW, RAW)],
                raw_sp.at[s, sid], rsems[s])

        def fire(w, s, gsem):
            pltpu.make_async_copy(
                idx_hbm.at[pl.ds(0, RAW)], raw_sp.at[s, sid], rsems[s]).wait()
            pltpu.sync_copy(raw_sp.at[s, sid], raw_s.at[s])
            base = base0 + w * W

            @pl.loop(0, W)
            def _(i):
                b0 = raw_s[s, 2 * W + i]
                b1 = raw_s[s, 3 * W + i]
                occ = (b0 + b1) > 0
                p0 = jnp.where(occ, PCODE_DIM, raw_s[s, i])
                p1 = jnp.where(occ, 2 * PCODE_DIM + 1,
                               raw_s[s, W + i] + (PCODE_DIM + 1))
                coff = ((base + i) % CELL_DIM) * EMBED_DIM
                pltpu.async_copy(
                    wp_hbm.at[pl.ds(p0, 1)], g_v.at[s, 0, pl.ds(i, 1)], gsem)
                pltpu.async_copy(
                    wp_hbm.at[pl.ds(p1, 1)], g_v.at[s, 1, pl.ds(i, 1)], gsem)
                pltpu.async_copy(
                    wb_hbm.at[pl.ds(coff + p0, 1)],
                    g_v.at[s, 2, pl.ds(i, 1)], gsem)
                pltpu.async_copy(
                    wb_hbm.at[pl.ds(coff + p1, 1)],
                    g_v.at[s, 3, pl.ds(i, 1)], gsem)

        def wait_gathers(s, gsem):
            @pl.loop(0, W)
            def _(i):
                for j in range(4):
                    pltpu.make_async_copy(
                        wp_hbm.at[pl.ds(0, 1)],
                        g_v.at[s, j, pl.ds(i, 1)], gsem
                    ).wait()

        def drain_out(s, osem):
            pltpu.make_async_copy(
                o_v.at[s], out_hbm.at[pl.ds(0, W)], osem
            ).wait()

        def accumulate(s):
            @pl.loop(0, W)
            def _(r):
                for c in range(FEATURE_DIM // LANES):
                    sl = pl.ds(c * LANES, LANES)
                    o_v[s, r, sl] = (g_v[s, 0, r, sl] + g_v[s, 1, r, sl]) + (
                        g_v[s, 2, r, sl] + g_v[s, 3, r, sl])

        def fire_out(w, s, osem):
            base = base0 + w * W
            pltpu.async_copy(o_v.at[s], out_hbm.at[pl.ds(base, W)], osem)

        def step(w, s, gsem, osem, drain_pred, last=False):
            wait_gathers(s, gsem)
            if drain_pred is True:
                drain_out(s, osem)
            elif drain_pred is not False:
                @pl.when(drain_pred)
                def _():
                    drain_out(s, osem)
            accumulate(s)
            fire_out(w, s, osem)
            if not last:
                @pl.when(w + 2 < wins)
                def _():
                    fire(w + 2, s, gsem)

                @pl.when(w + 4 < wins)
                def _():
                    stage(w + 4, s)

        stage(0, 0)
        stage(1, 1)
        fire(0, 0, gsA)
        fire(1, 1, gsB)
        stage(2, 0)
        stage(3, 1)

        @pl.loop(0, wins // 2)
        def _(t):
            step(2 * t, 0, gsA, osA, drain_pred=t > 0)
            step(2 * t + 1, 1, gsB, osB, drain_pred=t > 0)

        if wins % 2:
            step(wins - 1, 0, gsA, osA, drain_pred=True, last=True)
        drain_out(0, osA)
        drain_out(1, osB)

    return k(idx_in, w_pcode, w_board)


def _tc_transpose(s, batch):
    grp = 8

    def body(s_ref, o_ref):
        o_ref[...] = jnp.swapaxes(s_ref[...], 1, 2)

    return pl.pallas_call(
        body,
        grid=(batch // grp,),
        in_specs=[
            pl.BlockSpec((grp, CELL_DIM, FEATURE_DIM), lambda i: (i, 0, 0))
        ],
        out_specs=pl.BlockSpec((grp, FEATURE_DIM, CELL_DIM), lambda i: (i, 0, 0)),
        out_shape=jax.ShapeDtypeStruct((batch, FEATURE_DIM, CELL_DIM), jnp.float32),
    )(s)


def kernel(sparse_feature_input, sparse_feature_dim, board_input, W_pcode, W_board):
    del sparse_feature_dim
    batch = sparse_feature_input.shape[0]
    n_rows = batch * CELL_DIM
    pc = sparse_feature_input[:, 10:12].reshape(batch, 2, CELL_DIM)
    bd = board_input.reshape(batch, 2, CELL_DIM)
    idx_in = jnp.concatenate([pc, bd], axis=1)
    idx_flat = jnp.transpose(idx_in, (1, 0, 2)).reshape(4, n_rows)
    idx_win = jnp.transpose(
        idx_flat.reshape(4, n_rows // W, W), (1, 0, 2)
    ).reshape(-1)
    s = _sc_gather_sum(idx_win, W_pcode, W_board, n_rows)
    out = _tc_transpose(s.reshape(batch, CELL_DIM, FEATURE_DIM), batch)
    return out.reshape(batch, FEATURE_DIM, BOARD_SIZE, BOARD_SIZE)


# speedup vs baseline: 3.5687x; 3.5452x over previous
import functools

import jax
import jax.numpy as jnp
from jax import lax
from jax.experimental import pallas as pl
from jax.experimental.pallas import tpu as pltpu
from jax.experimental.pallas import tpu_sc as plsc

FEATURE_DIM = 128
BOARD_SIZE = 15
PCODE_DIM = 2380
EMBED_DIM = 2 * (PCODE_DIM + 1)
CELL_DIM = BOARD_SIZE * BOARD_SIZE

NUM_CORES = 2
NUM_SUBCORES = 16
LANES = 16

STRIPE = 304
REG_ROWS = 4776
CELLS0 = 112


def _sc_gather_sum(idx_in, w_pcode, w_board, batch):
    bpt = batch // NUM_SUBCORES
    rec = 4 * bpt
    n_board = CELL_DIM * EMBED_DIM
    mesh = plsc.VectorSubcoreMesh(core_axis_name="c", subcore_axis_name="s")

    @functools.partial(
        pl.kernel,
        mesh=mesh,
        out_type=jax.ShapeDtypeStruct((CELL_DIM, batch, FEATURE_DIM),
                                      jnp.float32),
        scratch_types=[
            pltpu.VMEM_SHARED((EMBED_DIM + 6, FEATURE_DIM), jnp.float32),
            pltpu.VMEM_SHARED((REG_ROWS, FEATURE_DIM), jnp.float32),
            pltpu.VMEM((4 * bpt,), jnp.int32),
            pltpu.VMEM((4, bpt), jnp.int32),
            pltpu.VMEM((4, bpt, FEATURE_DIM), jnp.float32),
            pltpu.VMEM((bpt, FEATURE_DIM), jnp.float32),
            pltpu.SemaphoreType.DMA,
            pltpu.SemaphoreType.DMA,
        ],
    )
    def k(idx_hbm, wp_hbm, wb_hbm, out_hbm, wp_sp, reg_sp, raw_v, idx_v,
          g_v, o_v, gsem, osem):
        cid = lax.axis_index("c")
        sid = lax.axis_index("s")

        @pl.when(sid < NUM_SUBCORES - 1)
        def _():
            off = pl.multiple_of(sid * STRIPE, 8)
            pltpu.sync_copy(wp_hbm.at[pl.ds(off, STRIPE)],
                            wp_sp.at[pl.ds(off, STRIPE)])

        @pl.when(sid == NUM_SUBCORES - 1)
        def _():
            tail = EMBED_DIM - 15 * STRIPE
            pltpu.sync_copy(wp_hbm.at[pl.ds(15 * STRIPE, tail)],
                            wp_sp.at[pl.ds(15 * STRIPE, tail)])

        c_lo = cid * CELLS0
        n_cells = CELLS0 + cid

        def do_cell(c, _):
            src0 = c * EMBED_DIM
            start = pl.multiple_of(src0 // 8 * 8, 8)
            skew = src0 - start

            @pl.when(sid < NUM_SUBCORES - 1)
            def _():
                off = pl.multiple_of(sid * STRIPE, 8)
                pltpu.sync_copy(wb_hbm.at[pl.ds(start + off, STRIPE)],
                                reg_sp.at[pl.ds(off, STRIPE)])

            @pl.when(sid == NUM_SUBCORES - 1)
            def _():
                off = 15 * STRIPE
                @pl.when(start + REG_ROWS <= n_board)
                def _():
                    pltpu.sync_copy(
                        wb_hbm.at[pl.ds(start + off, REG_ROWS - off)],
                        reg_sp.at[pl.ds(off, REG_ROWS - off)])

                @pl.when(start + REG_ROWS > n_board)
                def _():
                    pltpu.sync_copy(
                        wb_hbm.at[pl.ds(start + off, 192)],
                        reg_sp.at[pl.ds(off, 192)])

                    @pl.loop(0, 10)
                    def _(i):
                        pltpu.sync_copy(
                            wb_hbm.at[pl.ds(start + off + 192 + i, 1)],
                            reg_sp.at[pl.ds(off + 192 + i, 1)])

            plsc.subcore_barrier()

            pltpu.sync_copy(
                idx_hbm.at[pl.ds((c * NUM_SUBCORES + sid) * rec, rec)],
                raw_v)

            @pl.loop(0, bpt // LANES)
            def _(j):
                sl = pl.ds(j * LANES, LANES)
                occ = (raw_v[pl.ds(2 * bpt + j * LANES, LANES)]
                       + raw_v[pl.ds(3 * bpt + j * LANES, LANES)]) > 0
                p0 = jnp.where(occ, PCODE_DIM,
                               raw_v[pl.ds(j * LANES, LANES)])
                p1 = jnp.where(occ, 2 * PCODE_DIM + 1,
                               raw_v[pl.ds(bpt + j * LANES, LANES)]
                               + (PCODE_DIM + 1))
                idx_v[0, sl] = p0
                idx_v[1, sl] = p1
                idx_v[2, sl] = p0 + skew
                idx_v[3, sl] = p1 + skew

            pltpu.async_copy(wp_sp.at[idx_v.at[0]], g_v.at[0], gsem)
            pltpu.async_copy(wp_sp.at[idx_v.at[1]], g_v.at[1], gsem)
            pltpu.async_copy(reg_sp.at[idx_v.at[2]], g_v.at[2], gsem)
            pltpu.async_copy(reg_sp.at[idx_v.at[3]], g_v.at[3], gsem)
            for j, tab in enumerate((wp_sp, wp_sp, reg_sp, reg_sp)):
                pltpu.make_async_copy(
                    tab.at[idx_v.at[j]], g_v.at[j], gsem).wait()

            @pl.loop(0, bpt)
            def _(r):
                for q in range(FEATURE_DIM // LANES):
                    sl = pl.ds(q * LANES, LANES)
                    o_v[r, sl] = (g_v[0, r, sl] + g_v[1, r, sl]) + (
                        g_v[2, r, sl] + g_v[3, r, sl])

            boff = pl.multiple_of(sid * bpt, 8)
            pltpu.async_copy(o_v, out_hbm.at[c, pl.ds(boff, bpt)], osem)
            pltpu.make_async_copy(
                o_v, out_hbm.at[c, pl.ds(boff, bpt)], osem).wait()

            plsc.subcore_barrier()
            return 0

        plsc.subcore_barrier()
        lax.fori_loop(c_lo, c_lo + n_cells, do_cell, 0)

    return k(idx_in, w_pcode, w_board)


def _tc_transpose(s, batch):
    grp = 8

    def body(s_ref, o_ref):
        o_ref[...] = jnp.transpose(s_ref[...], (1, 2, 0))

    return pl.pallas_call(
        body,
        grid=(batch // grp,),
        in_specs=[
            pl.BlockSpec((CELL_DIM, grp, FEATURE_DIM), lambda i: (0, i, 0))
        ],
        out_specs=pl.BlockSpec((grp, FEATURE_DIM, CELL_DIM), lambda i: (i, 0, 0)),
        out_shape=jax.ShapeDtypeStruct((batch, FEATURE_DIM, CELL_DIM), jnp.float32),
    )(s)


def kernel(sparse_feature_input, sparse_feature_dim, board_input, W_pcode, W_board):
    del sparse_feature_dim
    batch = sparse_feature_input.shape[0]
    bpt = batch // NUM_SUBCORES
    pc = sparse_feature_input[:, 10:12].reshape(batch, 2, CELL_DIM)
    bd = board_input.reshape(batch, 2, CELL_DIM)
    raw = jnp.concatenate([pc, bd], axis=1)
    idx_in = jnp.transpose(raw, (2, 1, 0)).reshape(
        CELL_DIM, 4, NUM_SUBCORES, bpt)
    idx_in = jnp.transpose(idx_in, (0, 2, 1, 3)).reshape(-1)
    s = _sc_gather_sum(idx_in, W_pcode, W_board, batch)
    out = _tc_transpose(s, batch)
    return out.reshape(batch, FEATURE_DIM, BOARD_SIZE, BOARD_SIZE)


# speedup vs baseline: 4.2026x; 1.1776x over previous
import functools

import jax
import jax.numpy as jnp
from jax import lax
from jax.experimental import pallas as pl
from jax.experimental.pallas import tpu as pltpu
from jax.experimental.pallas import tpu_sc as plsc

FEATURE_DIM = 128
BOARD_SIZE = 15
PCODE_DIM = 2380
EMBED_DIM = 2 * (PCODE_DIM + 1)
CELL_DIM = BOARD_SIZE * BOARD_SIZE

NUM_CORES = 2
NUM_SUBCORES = 16
LANES = 16

STRIPE = 304
REG_ROWS = 4768
CELLS0 = 112
SG = 16


def _sc_gather_sum(idx_in, w_pcode, w_board, batch):
    bpt = batch // NUM_SUBCORES
    rec = 4 * bpt
    n_board = CELL_DIM * EMBED_DIM
    mesh = plsc.VectorSubcoreMesh(core_axis_name="c", subcore_axis_name="s")

    @functools.partial(
        pl.kernel,
        mesh=mesh,
        out_type=jax.ShapeDtypeStruct((CELL_DIM, batch, FEATURE_DIM),
                                      jnp.float32),
        scratch_types=[
            pltpu.VMEM_SHARED((EMBED_DIM, FEATURE_DIM), jnp.float32),
            pltpu.VMEM_SHARED((2, REG_ROWS, FEATURE_DIM), jnp.float32),
            pltpu.VMEM((rec,), jnp.int32),
            pltpu.VMEM((4, SG, FEATURE_DIM), jnp.float32),
            pltpu.SemaphoreType.DMA,
            pltpu.SemaphoreType.DMA,
            pltpu.SemaphoreType.DMA,
        ],
    )
    def k(idx_hbm, wp_hbm, wb_hbm, out_hbm, wp_sp, reg_sp, raw_v,
          g_v, gsem, osem, ssem):
        cid = lax.axis_index("c")
        sid = lax.axis_index("s")

        @pl.when(sid < NUM_SUBCORES - 1)
        def _():
            off = pl.multiple_of(sid * STRIPE, 8)
            pltpu.sync_copy(wp_hbm.at[pl.ds(off, STRIPE)],
                            wp_sp.at[pl.ds(off, STRIPE)])

        @pl.when(sid == NUM_SUBCORES - 1)
        def _():
            tail = EMBED_DIM - 15 * STRIPE
            pltpu.sync_copy(wp_hbm.at[pl.ds(15 * STRIPE, tail)],
                            wp_sp.at[pl.ds(15 * STRIPE, tail)])

        c_lo = cid * CELLS0
        n_cells = CELLS0 + cid

        def reg_start(c):
            src0 = c * EMBED_DIM
            start = pl.multiple_of(src0 // 8 * 8, 8)
            return start, src0 - start

        def stage_region(c, buf):
            start, _ = reg_start(c)

            @pl.when(sid < NUM_SUBCORES - 1)
            def _():
                off = pl.multiple_of(sid * STRIPE, 8)
                pltpu.async_copy(wb_hbm.at[pl.ds(start + off, STRIPE)],
                                 reg_sp.at[buf, pl.ds(off, STRIPE)], ssem)

            @pl.when(sid == NUM_SUBCORES - 1)
            def _():
                off = 15 * STRIPE
                @pl.when(start + REG_ROWS <= n_board)
                def _():
                    pltpu.async_copy(
                        wb_hbm.at[pl.ds(start + off, REG_ROWS - off)],
                        reg_sp.at[buf, pl.ds(off, REG_ROWS - off)], ssem)

                @pl.when(start + REG_ROWS > n_board)
                def _():
                    pltpu.async_copy(
                        wb_hbm.at[pl.ds(start + off, 192)],
                        reg_sp.at[buf, pl.ds(off, 192)], ssem)

                    @pl.loop(0, 10)
                    def _(i):
                        pltpu.sync_copy(
                            wb_hbm.at[pl.ds(start + off + 192 + i, 1)],
                            reg_sp.at[buf, pl.ds(off + 192 + i, 1)])

        def wait_region(c, buf):
            start, _ = reg_start(c)

            @pl.when(sid < NUM_SUBCORES - 1)
            def _():
                pltpu.make_async_copy(
                    wb_hbm.at[pl.ds(0, STRIPE)],
                    reg_sp.at[buf, pl.ds(0, STRIPE)], ssem).wait()

            @pl.when(sid == NUM_SUBCORES - 1)
            def _():
                off = 15 * STRIPE
                @pl.when(start + REG_ROWS <= n_board)
                def _():
                    pltpu.make_async_copy(
                        wb_hbm.at[pl.ds(0, REG_ROWS - off)],
                        reg_sp.at[buf, pl.ds(0, REG_ROWS - off)], ssem).wait()

                @pl.when(start + REG_ROWS > n_board)
                def _():
                    pltpu.make_async_copy(
                        wb_hbm.at[pl.ds(0, 192)],
                        reg_sp.at[buf, pl.ds(0, 192)], ssem).wait()

        def compute_cell(c, buf, next_c, next_buf, do_stage):
            _, skew = reg_start(c)
            reg = reg_sp.at[buf]

            pltpu.sync_copy(
                idx_hbm.at[pl.ds((c * NUM_SUBCORES + sid) * rec, rec)],
                raw_v)

            @pl.loop(0, bpt // LANES)
            def _(j):
                sl0 = pl.ds(j * LANES, LANES)
                sl1 = pl.ds(bpt + j * LANES, LANES)
                occ = (raw_v[pl.ds(2 * bpt + j * LANES, LANES)]
                       + raw_v[pl.ds(3 * bpt + j * LANES, LANES)]) > 0
                p0 = jnp.where(occ, PCODE_DIM, raw_v[sl0])
                p1 = jnp.where(occ, 2 * PCODE_DIM + 1,
                               raw_v[sl1] + (PCODE_DIM + 1))
                raw_v[sl0] = p0
                raw_v[sl1] = p1
                raw_v[pl.ds(2 * bpt + j * LANES, LANES)] = p0 + skew
                raw_v[pl.ds(3 * bpt + j * LANES, LANES)] = p1 + skew

            for g in range(bpt // SG):
                if g == 1 and do_stage is not None:
                    @pl.when(do_stage)
                    def _():
                        stage_region(next_c, next_buf)
                i0 = g * SG
                s0 = raw_v.at[pl.ds(i0, SG)]
                s1 = raw_v.at[pl.ds(bpt + i0, SG)]
                s2 = raw_v.at[pl.ds(2 * bpt + i0, SG)]
                s3 = raw_v.at[pl.ds(3 * bpt + i0, SG)]
                pltpu.async_copy(wp_sp.at[s0], g_v.at[0], gsem)
                pltpu.async_copy(wp_sp.at[s1], g_v.at[1], gsem)
                pltpu.async_copy(reg.at[s2], g_v.at[2], gsem)
                pltpu.async_copy(reg.at[s3], g_v.at[3], gsem)
                for j, tab in enumerate((wp_sp.at[s0], wp_sp.at[s1],
                                         reg.at[s2], reg.at[s3])):
                    pltpu.make_async_copy(tab, g_v.at[j], gsem).wait()

                @pl.loop(0, SG)
                def _(r):
                    for q in range(FEATURE_DIM // LANES):
                        sl = pl.ds(q * LANES, LANES)
                        g_v[3, r, sl] = (g_v[0, r, sl] + g_v[1, r, sl]) + (
                            g_v[2, r, sl] + g_v[3, r, sl])

                boff = pl.multiple_of(sid * bpt + i0, 8)
                pltpu.async_copy(
                    g_v.at[3], out_hbm.at[c, pl.ds(boff, SG)], osem)

            for g in range(bpt // SG):
                boff = pl.multiple_of(sid * bpt + g * SG, 8)
                pltpu.make_async_copy(
                    g_v.at[3], out_hbm.at[c, pl.ds(boff, SG)], osem).wait()

        plsc.subcore_barrier()

        stage_region(c_lo, 0)
        wait_region(c_lo, 0)
        plsc.subcore_barrier()

        n_pairs = n_cells // 2

        def do_pair(t, _):
            a = c_lo + 2 * t
            compute_cell(a, 0, a + 1, 1, do_stage=a + 1 < c_lo + n_cells)
            wait_region(a + 1, 1)
            plsc.subcore_barrier()
            more = a + 2 < c_lo + n_cells
            compute_cell(a + 1, 1, a + 2, 0, do_stage=more)

            @pl.when(more)
            def _():
                wait_region(a + 2, 0)

            plsc.subcore_barrier()
            return 0

        lax.fori_loop(0, n_pairs, do_pair, 0)

        @pl.when(n_cells % 2 == 1)
        def _():
            compute_cell(c_lo + n_cells - 1, 0, 0, 1, do_stage=None)

    return k(idx_in, w_pcode, w_board)


def _tc_transpose(s, batch):
    grp = 8

    def body(s_ref, o_ref):
        o_ref[...] = jnp.transpose(s_ref[...], (1, 2, 0))

    return pl.pallas_call(
        body,
        grid=(batch // grp,),
        in_specs=[
            pl.BlockSpec((CELL_DIM, grp, FEATURE_DIM), lambda i: (0, i, 0))
        ],
        out_specs=pl.BlockSpec((grp, FEATURE_DIM, CELL_DIM), lambda i: (i, 0, 0)),
        out_shape=jax.ShapeDtypeStruct((batch, FEATURE_DIM, CELL_DIM), jnp.float32),
    )(s)


def kernel(sparse_feature_input, sparse_feature_dim, board_input, W_pcode, W_board):
    del sparse_feature_dim
    batch = sparse_feature_input.shape[0]
    bpt = batch // NUM_SUBCORES
    pc = sparse_feature_input[:, 10:12].reshape(batch, 2, CELL_DIM)
    bd = board_input.reshape(batch, 2, CELL_DIM)
    raw = jnp.concatenate([pc, bd], axis=1)
    idx_in = jnp.transpose(raw, (2, 1, 0)).reshape(
        CELL_DIM, 4, NUM_SUBCORES, bpt)
    idx_in = jnp.transpose(idx_in, (0, 2, 1, 3)).reshape(-1)
    s = _sc_gather_sum(idx_in, W_pcode, W_board, batch)
    out = _tc_transpose(s, batch)
    return out.reshape(batch, FEATURE_DIM, BOARD_SIZE, BOARD_SIZE)
